# baseline (speedup 1.0000x reference)
import jax
import jax.numpy as jnp
from jax import lax
from jax.experimental import pallas as pl
from jax.experimental.pallas import tpu as pltpu
from jax.experimental.pallas import tpu_sc as plsc

NC, NS_SC, L = 2, 16, 16
NW = NC * NS_SC
R = 512
SL = 128
PTW = 256
AL = 8
RPW = R // NW
RPS = R // NS_SC


def _dpl_body(ts_hbm, asp_hbm, pt_hbm,
              out_ts, out_a, out_pt, out_g,
              ts_blk, mask_blk, mask_sh, mask_all, idx_all, sums_v, offs_v,
              myidx, g_vmem, asp_v, a_stage, ts_rows, pt_rows, sem):
    cid = lax.axis_index("c")
    sid = lax.axis_index("s")
    wid = cid * NS_SC + sid
    iota = lax.iota(jnp.int32, L)

    pltpu.sync_copy(ts_hbm.at[pl.ds(sid * RPS, RPS)], ts_blk)
    pltpu.sync_copy(asp_hbm, asp_v)
    accs = []
    for g in range(RPS // L):
        rows = iota + g * L
        par = [plsc.load_gather(ts_blk, [rows, jnp.full((L,), c, jnp.int32)])
               for c in range(4)]
        for c in range(4, SL):
            j = c & 3
            par[j] = par[j] | plsc.load_gather(
                ts_blk, [rows, jnp.full((L,), c, jnp.int32)])
        accs.append((par[0] | par[1]) | (par[2] | par[3]))
    for g in range(RPS // L):
        mask_blk[pl.ds(g * L, L)] = (accs[g] != 0).astype(jnp.int32)
    pltpu.sync_copy(mask_blk, mask_sh.at[pl.ds(sid * RPS, RPS)])
    plsc.subcore_barrier()

    pltpu.sync_copy(mask_sh, mask_all)
    zero = jnp.zeros((L,), jnp.int32)
    for k in range(R // L):
        idx_all[pl.ds(k * L, L)] = zero
    lane0 = iota == 0
    chunks = [mask_all[pl.ds(k * L, L)] for k in range(R // L)]
    for k in range(R // L):
        s = jnp.broadcast_to(jnp.sum(chunks[k]), (L,))
        plsc.store_scatter(sums_v, [jnp.full((L,), k, jnp.int32)], s,
                           mask=lane0)
    s0 = sums_v[pl.ds(0, L)]
    s1 = sums_v[pl.ds(L, L)]
    offs_v[pl.ds(0, L)] = plsc.cumsum(s0) - s0
    offs_v[pl.ds(L, L)] = plsc.cumsum(s1) - s1 + jnp.sum(s0)
    for k in range(R // L):
        m = chunks[k]
        offk = plsc.load_gather(offs_v, [jnp.full((L,), k, jnp.int32)])
        pos = plsc.cumsum(m) + offk - 1
        plsc.store_scatter(idx_all, [pos], iota + k * L, mask=(m != 0))

    base = wid * RPW
    idx_vec = idx_all[pl.ds(base, RPW)]
    myidx[...] = idx_vec
    g_vmem[...] = lax.shift_right_logical(idx_vec, 5)

    cp_ts = pltpu.async_copy(ts_hbm.at[myidx], ts_rows, sem)
    cp_pt = pltpu.async_copy(pt_hbm.at[myidx], pt_rows, sem)

    for k in range(RPW * AL // L):
        p = iota + k * L
        r = lax.shift_right_logical(p, 3)
        c = jnp.bitwise_and(p, 7)
        gr = plsc.load_gather(g_vmem, [r])
        av = plsc.load_gather(asp_v, [gr * AL + c])
        plsc.store_scatter(a_stage, [p], av)

    cp_ts.wait()
    cp_pt.wait()
    pltpu.sync_copy(ts_rows, out_ts.at[pl.ds(base, RPW)])
    pltpu.sync_copy(pt_rows, out_pt.at[pl.ds(base, RPW)])
    pltpu.sync_copy(a_stage, out_a.at[pl.ds(base * AL, RPW * AL)])
    pltpu.sync_copy(g_vmem, out_g.at[pl.ds(base, RPW)])


@jax.jit
def _dpl_call(ts2, asp, pt2):
    f = pl.kernel(
        _dpl_body,
        out_type=(
            jax.ShapeDtypeStruct((R, SL), jnp.int32),
            jax.ShapeDtypeStruct((R * AL,), jnp.int32),
            jax.ShapeDtypeStruct((R, PTW), jnp.float32),
            jax.ShapeDtypeStruct((R,), jnp.int32),
        ),
        mesh=plsc.VectorSubcoreMesh(core_axis_name="c", subcore_axis_name="s"),
        compiler_params=pltpu.CompilerParams(needs_layout_passes=False),
        scratch_types=[
            pltpu.VMEM((RPS, SL), jnp.int32),
            pltpu.VMEM((RPS,), jnp.int32),
            pltpu.VMEM_SHARED((R,), jnp.int32),
            pltpu.VMEM((R,), jnp.int32),
            pltpu.VMEM((R,), jnp.int32),
            pltpu.VMEM((2 * L,), jnp.int32),
            pltpu.VMEM((2 * L,), jnp.int32),
            pltpu.VMEM((RPW,), jnp.int32),
            pltpu.VMEM((RPW,), jnp.int32),
            pltpu.VMEM((16 * AL,), jnp.int32),
            pltpu.VMEM((RPW * AL,), jnp.int32),
            pltpu.VMEM((RPW, SL), jnp.int32),
            pltpu.VMEM((RPW, PTW), jnp.float32),
            pltpu.SemaphoreType.DMA,
        ],
    )
    return f(ts2, asp, pt2)


def kernel(text_slices, aspect_tokens, pos_tuple):
    b, ns, sl = text_slices.shape
    ts2 = text_slices.reshape(b * ns, sl).astype(jnp.int32)
    pt2 = pos_tuple.reshape(b * ns, sl * 2)
    asp = aspect_tokens.astype(jnp.int32).reshape(-1)
    ts_sel, a_sel, pt_sel, g_sel = _dpl_call(ts2, asp, pt2)
    return (ts_sel, a_sel.reshape(b * ns, aspect_tokens.shape[1]),
            pt_sel.reshape(b * ns, sl, 2), g_sel)


# speedup vs baseline: 1.0419x; 1.0419x over previous
import jax
import jax.numpy as jnp
from jax import lax
from jax.experimental import pallas as pl
from jax.experimental.pallas import tpu as pltpu
from jax.experimental.pallas import tpu_sc as plsc

NC, NS_SC, L = 2, 16, 16
NW = NC * NS_SC
R = 512
SL = 128
PTW = 256
AL = 8
RPW = R // NW
RPS = R // NS_SC


def _dpl_body(ts_hbm, asp_hbm, pt_hbm,
              out_ts, out_a, out_pt, out_g,
              ts_blk, mask_blk, mask_sh, mask_all, idx_all, sums_v, offs_v,
              myidx, g_vmem, asp_v, a_stage, ts_rows, pt_rows, sem):
    cid = lax.axis_index("c")
    sid = lax.axis_index("s")
    wid = cid * NS_SC + sid
    iota = lax.iota(jnp.int32, L)

    pltpu.sync_copy(ts_hbm.at[pl.ds(sid * RPS, RPS)], ts_blk)
    pltpu.sync_copy(asp_hbm, asp_v)
    accs = []
    for g in range(RPS // L):
        rows = iota + g * L
        par = [plsc.load_gather(ts_blk, [rows, jnp.full((L,), c, jnp.int32)])
               for c in range(4)]
        for c in range(4, SL):
            j = c & 3
            par[j] = par[j] | plsc.load_gather(
                ts_blk, [rows, jnp.full((L,), c, jnp.int32)])
        accs.append((par[0] | par[1]) | (par[2] | par[3]))
    for g in range(RPS // L):
        mask_blk[pl.ds(g * L, L)] = (accs[g] != 0).astype(jnp.int32)
    pltpu.sync_copy(mask_blk, mask_sh.at[pl.ds(sid * RPS, RPS)])
    plsc.subcore_barrier()

    pltpu.sync_copy(mask_sh, mask_all)
    zero = jnp.zeros((L,), jnp.int32)
    for k in range(R // L):
        idx_all[pl.ds(k * L, L)] = zero
    carry = jnp.int32(0)
    for k in range(R // L):
        m = mask_all[pl.ds(k * L, L)]
        cs = plsc.cumsum(m)
        pos = cs + carry - 1
        plsc.store_scatter(idx_all, [pos], iota + k * L, mask=(m != 0))
        carry = carry + jnp.sum(m)

    base = wid * RPW
    idx_vec = idx_all[pl.ds(base, RPW)]
    myidx[...] = idx_vec
    g_vmem[...] = lax.shift_right_logical(idx_vec, 5)

    cp_ts = pltpu.async_copy(ts_hbm.at[myidx], ts_rows, sem)
    cp_pt = pltpu.async_copy(pt_hbm.at[myidx], pt_rows, sem)

    for k in range(RPW * AL // L):
        p = iota + k * L
        r = lax.shift_right_logical(p, 3)
        c = jnp.bitwise_and(p, 7)
        gr = plsc.load_gather(g_vmem, [r])
        av = plsc.load_gather(asp_v, [gr * AL + c])
        plsc.store_scatter(a_stage, [p], av)

    cp_ts.wait()
    cp_pt.wait()
    pltpu.sync_copy(ts_rows, out_ts.at[pl.ds(base, RPW)])
    pltpu.sync_copy(pt_rows, out_pt.at[pl.ds(base, RPW)])
    pltpu.sync_copy(a_stage, out_a.at[pl.ds(base * AL, RPW * AL)])
    pltpu.sync_copy(g_vmem, out_g.at[pl.ds(base, RPW)])


@jax.jit
def _dpl_call(ts2, asp, pt2):
    f = pl.kernel(
        _dpl_body,
        out_type=(
            jax.ShapeDtypeStruct((R, SL), jnp.int32),
            jax.ShapeDtypeStruct((R * AL,), jnp.int32),
            jax.ShapeDtypeStruct((R, PTW), jnp.float32),
            jax.ShapeDtypeStruct((R,), jnp.int32),
        ),
        mesh=plsc.VectorSubcoreMesh(core_axis_name="c", subcore_axis_name="s"),
        compiler_params=pltpu.CompilerParams(needs_layout_passes=False),
        scratch_types=[
            pltpu.VMEM((RPS, SL), jnp.int32),
            pltpu.VMEM((RPS,), jnp.int32),
            pltpu.VMEM_SHARED((R,), jnp.int32),
            pltpu.VMEM((R,), jnp.int32),
            pltpu.VMEM((R,), jnp.int32),
            pltpu.VMEM((2 * L,), jnp.int32),
            pltpu.VMEM((2 * L,), jnp.int32),
            pltpu.VMEM((RPW,), jnp.int32),
            pltpu.VMEM((RPW,), jnp.int32),
            pltpu.VMEM((16 * AL,), jnp.int32),
            pltpu.VMEM((RPW * AL,), jnp.int32),
            pltpu.VMEM((RPW, SL), jnp.int32),
            pltpu.VMEM((RPW, PTW), jnp.float32),
            pltpu.SemaphoreType.DMA,
        ],
    )
    return f(ts2, asp, pt2)


def kernel(text_slices, aspect_tokens, pos_tuple):
    b, ns, sl = text_slices.shape
    ts2 = text_slices.reshape(b * ns, sl).astype(jnp.int32)
    pt2 = pos_tuple.reshape(b * ns, sl * 2)
    asp = aspect_tokens.astype(jnp.int32).reshape(-1)
    ts_sel, a_sel, pt_sel, g_sel = _dpl_call(ts2, asp, pt2)
    return (ts_sel, a_sel.reshape(b * ns, aspect_tokens.shape[1]),
            pt_sel.reshape(b * ns, sl, 2), g_sel)


# speedup vs baseline: 1.0776x; 1.0343x over previous
import jax
import jax.numpy as jnp
from jax import lax
from jax.experimental import pallas as pl
from jax.experimental.pallas import tpu as pltpu
from jax.experimental.pallas import tpu_sc as plsc

NC, NS_SC, L = 2, 16, 16
NW = NC * NS_SC
R = 512
SL = 128
PTW = 256
AL = 8
RPW = R // NW
RPS = R // NS_SC


def _dpl_body(ts_hbm, asp_hbm, pt_hbm,
              out_ts, out_a, out_pt, out_g,
              ts_blk, mask_blk, mask_sh, mask_all, idx_all,
              myidx, g_vmem, asp_v, a_stage, ts_rows, pt_rows, sem):
    cid = lax.axis_index("c")
    sid = lax.axis_index("s")
    wid = cid * NS_SC + sid
    iota = lax.iota(jnp.int32, L)

    pltpu.sync_copy(ts_hbm.at[pl.ds(sid * RPS, RPS)], ts_blk)
    pltpu.sync_copy(asp_hbm, asp_v)

    def mask_step(c, accs):
        col = jnp.full((L,), c, jnp.int32)
        return (accs[0] | plsc.load_gather(ts_blk, [iota, col]),
                accs[1] | plsc.load_gather(ts_blk, [iota + L, col]))

    zero = jnp.zeros((L,), jnp.int32)
    acc0, acc1 = lax.fori_loop(0, SL, mask_step, (zero, zero))
    mask_blk[pl.ds(0, L)] = (acc0 != 0).astype(jnp.int32)
    mask_blk[pl.ds(L, L)] = (acc1 != 0).astype(jnp.int32)
    pltpu.sync_copy(mask_blk, mask_sh.at[pl.ds(sid * RPS, RPS)])
    plsc.subcore_barrier()

    pltpu.sync_copy(mask_sh, mask_all)

    def init_step(k, carry):
        idx_all[pl.ds(k * L, L)] = zero
        return carry

    lax.fori_loop(0, R // L, init_step, 0)

    def scan_step(k, carry):
        m = mask_all[pl.ds(k * L, L)]
        cs = plsc.cumsum(m)
        plsc.store_scatter(idx_all, [cs + carry - 1], iota + k * L,
                           mask=(m != 0))
        return carry + jnp.sum(m)

    lax.fori_loop(0, R // L, scan_step, jnp.int32(0))

    base = wid * RPW
    idx_vec = idx_all[pl.ds(base, RPW)]
    myidx[...] = idx_vec
    g_vmem[...] = lax.shift_right_logical(idx_vec, 5)

    cp_ts = pltpu.async_copy(ts_hbm.at[myidx], ts_rows, sem)
    cp_pt = pltpu.async_copy(pt_hbm.at[myidx], pt_rows, sem)

    def asp_step(k, carry):
        p = iota + k * L
        r = lax.shift_right_logical(p, 3)
        c = jnp.bitwise_and(p, 7)
        gr = plsc.load_gather(g_vmem, [r])
        av = plsc.load_gather(asp_v, [gr * AL + c])
        plsc.store_scatter(a_stage, [p], av)
        return carry

    lax.fori_loop(0, RPW * AL // L, asp_step, 0)

    cp_ts.wait()
    cp_pt.wait()
    pltpu.sync_copy(ts_rows, out_ts.at[pl.ds(base, RPW)])
    pltpu.sync_copy(pt_rows, out_pt.at[pl.ds(base, RPW)])
    pltpu.sync_copy(a_stage, out_a.at[pl.ds(base * AL, RPW * AL)])
    pltpu.sync_copy(g_vmem, out_g.at[pl.ds(base, RPW)])


@jax.jit
def _dpl_call(ts2, asp, pt2):
    f = pl.kernel(
        _dpl_body,
        out_type=(
            jax.ShapeDtypeStruct((R, SL), jnp.int32),
            jax.ShapeDtypeStruct((R * AL,), jnp.int32),
            jax.ShapeDtypeStruct((R, PTW), jnp.float32),
            jax.ShapeDtypeStruct((R,), jnp.int32),
        ),
        mesh=plsc.VectorSubcoreMesh(core_axis_name="c", subcore_axis_name="s"),
        compiler_params=pltpu.CompilerParams(needs_layout_passes=False),
        scratch_types=[
            pltpu.VMEM((RPS, SL), jnp.int32),
            pltpu.VMEM((RPS,), jnp.int32),
            pltpu.VMEM_SHARED((R,), jnp.int32),
            pltpu.VMEM((R,), jnp.int32),
            pltpu.VMEM((R,), jnp.int32),
            pltpu.VMEM((RPW,), jnp.int32),
            pltpu.VMEM((RPW,), jnp.int32),
            pltpu.VMEM((16 * AL,), jnp.int32),
            pltpu.VMEM((RPW * AL,), jnp.int32),
            pltpu.VMEM((RPW, SL), jnp.int32),
            pltpu.VMEM((RPW, PTW), jnp.float32),
            pltpu.SemaphoreType.DMA,
        ],
    )
    return f(ts2, asp, pt2)


def kernel(text_slices, aspect_tokens, pos_tuple):
    b, ns, sl = text_slices.shape
    ts2 = text_slices.reshape(b * ns, sl).astype(jnp.int32)
    pt2 = pos_tuple.reshape(b * ns, sl * 2)
    asp = aspect_tokens.astype(jnp.int32).reshape(-1)
    ts_sel, a_sel, pt_sel, g_sel = _dpl_call(ts2, asp, pt2)
    return (ts_sel, a_sel.reshape(b * ns, aspect_tokens.shape[1]),
            pt_sel.reshape(b * ns, sl, 2), g_sel)
